# Initial kernel scaffold; baseline (speedup 1.0000x reference)
import jax
import jax.numpy as jnp
from jax.experimental import pallas as pl


def kernel(weight, wgt_alpha):
    raise NotImplementedError("write your pallas kernel here")


# speedup vs baseline: 2.0300x; 2.0300x over previous
import functools

import jax
import jax.numpy as jnp
from jax import lax
from jax.experimental import pallas as pl
from jax.experimental.pallas import tpu as pltpu
from jax.experimental.pallas import tpu_sc as plsc

NC = 2
NS = 16
L = 16
NW = NC * NS
R, C = 2048, 2048
N = R * C
PER_W = N // NW
CHUNK = 16384
NCHUNK = PER_W // CHUNK
VPC = CHUNK // L

_MESH = plsc.VectorSubcoreMesh(
    core_axis_name="c", subcore_axis_name="s", num_cores=NC, num_subcores=NS
)


def _wid():
    return lax.axis_index("s") * NC + lax.axis_index("c")


def _lane_sum(v, tmp):
    lanes = lax.iota(jnp.int32, L)
    for s in (1, 2, 4, 8):
        tmp[...] = v
        v = v + plsc.load_gather(tmp, [lanes ^ s])
    return v


def _reduce_body(w_hbm, psum_hbm, psq_hbm, buf, svec, qvec):
    base = _wid() * PER_W

    def chunk_body(c, carry):
        s, q = carry
        pltpu.sync_copy(w_hbm.at[pl.ds(base + c * CHUNK, CHUNK)], buf)

        def vec_body(i, carry2):
            s2, q2 = carry2
            x = buf[pl.ds(i * L, L)]
            return s2 + x, q2 + x * x

        return lax.fori_loop(0, VPC, vec_body, (s, q))

    zero = jnp.zeros((L,), jnp.float32)
    s, q = lax.fori_loop(0, NCHUNK, chunk_body, (zero, zero))
    svec[...] = s
    qvec[...] = q
    off = _wid() * L
    pltpu.sync_copy(svec, psum_hbm.at[pl.ds(off, L)])
    pltpu.sync_copy(qvec, psq_hbm.at[pl.ds(off, L)])


@functools.partial(
    pl.kernel,
    out_type=(
        jax.ShapeDtypeStruct((NW * L,), jnp.float32),
        jax.ShapeDtypeStruct((NW * L,), jnp.float32),
    ),
    mesh=_MESH,
    compiler_params=pltpu.CompilerParams(needs_layout_passes=False),
    scratch_types=[
        pltpu.VMEM((CHUNK,), jnp.float32),
        pltpu.VMEM((L,), jnp.float32),
        pltpu.VMEM((L,), jnp.float32),
    ],
)
def _reduce_call(w_hbm, psum_hbm, psq_hbm, buf, svec, qvec):
    _reduce_body(w_hbm, psum_hbm, psq_hbm, buf, svec, qvec)


def _xform_body(w_hbm, psum_hbm, psq_hbm, grid_hbm, alpha_hbm, out_hbm,
                pbuf, grid_v, alpha_v, tmp_v, in_buf, out_buf):
    base = _wid() * PER_W

    pltpu.sync_copy(psum_hbm, pbuf.at[pl.ds(0, NW * L)])
    pltpu.sync_copy(psq_hbm, pbuf.at[pl.ds(NW * L, NW * L)])
    pltpu.sync_copy(grid_hbm, grid_v)
    pltpu.sync_copy(alpha_hbm, alpha_v)

    def acc_body(i, carry):
        s, q = carry
        return (s + pbuf[pl.ds(i * L, L)],
                q + pbuf[pl.ds(NW * L + i * L, L)])

    z = jnp.zeros((L,), jnp.float32)
    s, q = lax.fori_loop(0, NW, acc_body, (z, z))
    tot = _lane_sum(s, tmp_v)
    totq = _lane_sum(q, tmp_v)
    mean = tot * jnp.float32(1.0 / N)
    var = (totq - jnp.float32(N) * mean * mean) * jnp.float32(1.0 / (N - 1))
    vb = plsc.bitcast(var, jnp.int32)
    magic = jnp.full((L,), 0x5F3759DF, dtype=jnp.int32)
    y = plsc.bitcast(magic - lax.shift_right_logical(vb, 1), jnp.float32)
    for _ in range(3):
        y = y * (jnp.float32(1.5) - jnp.float32(0.5) * var * y * y)
    alpha = alpha_v[...]
    scale = y / alpha

    def chunk_body(c, carry):
        pltpu.sync_copy(w_hbm.at[pl.ds(base + c * CHUNK, CHUNK)], in_buf)

        def vec_body(i, carry2):
            x = in_buf[pl.ds(i * L, L)]
            yv = (x - mean) * scale
            sg = jnp.sign(yv)
            a = jnp.minimum(jnp.abs(yv), 1.0)
            k = (7.5 - a * 7.0).astype(jnp.int32)
            idx = 7 - k
            g = plsc.load_gather(grid_v, [idx])
            out_buf[pl.ds(i * L, L)] = g * (sg * alpha)
            return carry2

        lax.fori_loop(0, VPC, vec_body, 0)
        pltpu.sync_copy(out_buf, out_hbm.at[pl.ds(base + c * CHUNK, CHUNK)])
        return carry

    lax.fori_loop(0, NCHUNK, chunk_body, 0)


@functools.partial(
    pl.kernel,
    out_type=jax.ShapeDtypeStruct((N,), jnp.float32),
    mesh=_MESH,
    compiler_params=pltpu.CompilerParams(needs_layout_passes=False),
    scratch_types=[
        pltpu.VMEM((2 * NW * L,), jnp.float32),
        pltpu.VMEM((L,), jnp.float32),
        pltpu.VMEM((L,), jnp.float32),
        pltpu.VMEM((L,), jnp.float32),
        pltpu.VMEM((CHUNK,), jnp.float32),
        pltpu.VMEM((CHUNK,), jnp.float32),
    ],
)
def _xform_call(w_hbm, psum_hbm, psq_hbm, grid_hbm, alpha_hbm, out_hbm,
                pbuf, grid_v, alpha_v, tmp_v, in_buf, out_buf):
    _xform_body(w_hbm, psum_hbm, psq_hbm, grid_hbm, alpha_hbm, out_hbm,
                pbuf, grid_v, alpha_v, tmp_v, in_buf, out_buf)


def kernel(weight, wgt_alpha):
    w = weight.reshape(N)
    grid = jnp.linspace(0.0, 1.0, 8, dtype=jnp.float32) * 1.0
    grid16 = jnp.concatenate([grid, jnp.zeros((8,), jnp.float32)])
    alpha16 = jnp.full((L,), wgt_alpha, dtype=jnp.float32)
    psum, psq = _reduce_call(w)
    out = _xform_call(w, psum, psq, grid16, alpha16)
    return out.reshape(R, C)


# speedup vs baseline: 2.8641x; 1.4109x over previous
import functools

import jax
import jax.numpy as jnp
from jax import lax
from jax.experimental import pallas as pl
from jax.experimental.pallas import tpu as pltpu
from jax.experimental.pallas import tpu_sc as plsc

NC = 2
NS = 16
L = 16
NW = NC * NS
R, C = 2048, 2048
N = R * C
PER_W = N // NW
CHUNK = 16384
NCHUNK = PER_W // CHUNK
VPC = CHUNK // L
U = 8

_ABS_MASK = 0x7FFFFFFF
_SIGN_MASK = -0x80000000

_MESH = plsc.VectorSubcoreMesh(
    core_axis_name="c", subcore_axis_name="s", num_cores=NC, num_subcores=NS
)


def _wid():
    return lax.axis_index("s") * NC + lax.axis_index("c")


def _lane_sum(v, tmp):
    lanes = lax.iota(jnp.int32, L)
    for s in (1, 2, 4, 8):
        tmp[...] = v
        v = v + plsc.load_gather(tmp, [lanes ^ s])
    return v


def _in_copy(w_hbm, base, c, buf, sem):
    return pltpu.make_async_copy(
        w_hbm.at[pl.ds(base + c * CHUNK, CHUNK)], buf, sem
    )


def _reduce_body(w_hbm, psum_hbm, psq_hbm, buf0, buf1, svec, qvec, sem0, sem1):
    base = _wid() * PER_W
    _in_copy(w_hbm, base, 0, buf0, sem0).start()
    _in_copy(w_hbm, base, 1, buf1, sem1).start()

    def _acc(buf, carry):
        def vec_body(i, carry2):
            s2, q2 = carry2
            xs = [buf[pl.ds((i * U + u) * L, L)] for u in range(U)]
            for u in range(0, U, 2):
                s2 = s2 + (xs[u] + xs[u + 1])
                q2 = q2 + (xs[u] * xs[u] + xs[u + 1] * xs[u + 1])
            return s2, q2

        return lax.fori_loop(0, VPC // U, vec_body, carry)

    def chunk2(j2, carry):
        c0 = 2 * j2
        _in_copy(w_hbm, base, c0, buf0, sem0).wait()
        carry = _acc(buf0, carry)

        @pl.when(c0 + 2 < NCHUNK)
        def _():
            _in_copy(w_hbm, base, c0 + 2, buf0, sem0).start()

        _in_copy(w_hbm, base, c0 + 1, buf1, sem1).wait()
        carry = _acc(buf1, carry)

        @pl.when(c0 + 3 < NCHUNK)
        def _():
            _in_copy(w_hbm, base, c0 + 3, buf1, sem1).start()

        return carry

    zero = jnp.zeros((L,), jnp.float32)
    s, q = lax.fori_loop(0, NCHUNK // 2, chunk2, (zero, zero))
    svec[...] = s
    qvec[...] = q
    off = _wid() * L
    pltpu.sync_copy(svec, psum_hbm.at[pl.ds(off, L)])
    pltpu.sync_copy(qvec, psq_hbm.at[pl.ds(off, L)])


@functools.partial(
    pl.kernel,
    out_type=(
        jax.ShapeDtypeStruct((NW * L,), jnp.float32),
        jax.ShapeDtypeStruct((NW * L,), jnp.float32),
    ),
    mesh=_MESH,
    compiler_params=pltpu.CompilerParams(needs_layout_passes=False),
    scratch_types=[
        pltpu.VMEM((CHUNK,), jnp.float32),
        pltpu.VMEM((CHUNK,), jnp.float32),
        pltpu.VMEM((L,), jnp.float32),
        pltpu.VMEM((L,), jnp.float32),
        pltpu.SemaphoreType.DMA,
        pltpu.SemaphoreType.DMA,
    ],
)
def _reduce_call(w_hbm, psum_hbm, psq_hbm, buf0, buf1, svec, qvec, sem0, sem1):
    _reduce_body(w_hbm, psum_hbm, psq_hbm, buf0, buf1, svec, qvec, sem0, sem1)


def _xform_body(w_hbm, psum_hbm, psq_hbm, table_hbm, alpha_hbm, out_hbm,
                pbuf, table_v, alpha_v, tmp_v,
                in0, in1, out0, out1, si0, si1, so0, so1):
    base = _wid() * PER_W
    _in_copy(w_hbm, base, 0, in0, si0).start()
    _in_copy(w_hbm, base, 1, in1, si1).start()

    pltpu.sync_copy(psum_hbm, pbuf.at[pl.ds(0, NW * L)])
    pltpu.sync_copy(psq_hbm, pbuf.at[pl.ds(NW * L, NW * L)])
    pltpu.sync_copy(table_hbm, table_v)
    pltpu.sync_copy(alpha_hbm, alpha_v)

    def acc_body(i, carry):
        s, q = carry
        return (s + pbuf[pl.ds(i * L, L)],
                q + pbuf[pl.ds(NW * L + i * L, L)])

    z16 = jnp.zeros((L,), jnp.float32)
    s, q = lax.fori_loop(0, NW, acc_body, (z16, z16))
    tot = _lane_sum(s, tmp_v)
    totq = _lane_sum(q, tmp_v)
    mean = tot * jnp.float32(1.0 / N)
    var = (totq - jnp.float32(N) * mean * mean) * jnp.float32(1.0 / (N - 1))
    vb = plsc.bitcast(var, jnp.int32)
    magic = jnp.full((L,), 0x5F3759DF, dtype=jnp.int32)
    y = plsc.bitcast(magic - lax.shift_right_logical(vb, 1), jnp.float32)
    for _ in range(3):
        y = y * (jnp.float32(1.5) - jnp.float32(0.5) * var * y * y)
    s7 = (y / alpha_v[...]) * jnp.float32(7.0)
    m7 = mean * s7

    abs_mask = jnp.full((L,), _ABS_MASK, dtype=jnp.int32)
    sign_mask = jnp.full((L,), _SIGN_MASK, dtype=jnp.int32)
    seven_i = jnp.full((L,), 7, dtype=jnp.int32)
    seven_f = jnp.full((L,), 7.0, dtype=jnp.float32)
    half8 = jnp.full((L,), 7.5, dtype=jnp.float32)

    def _out_copy(c, buf, sem):
        return pltpu.make_async_copy(
            buf, out_hbm.at[pl.ds(base + c * CHUNK, CHUNK)], sem
        )

    def _xf(ibuf, obuf):
        def vec_body(i, carry2):
            for u in range(U):
                o = (i * U + u) * L
                x = ibuf[pl.ds(o, L)]
                z = x * s7 - m7
                zi = plsc.bitcast(z, jnp.int32)
                az = plsc.bitcast(zi & abs_mask, jnp.float32)
                az = jnp.minimum(az, seven_f)
                idx = seven_i - (half8 - az).astype(jnp.int32)
                g = plsc.load_gather(table_v, [idx])
                gi = plsc.bitcast(g, jnp.int32) ^ (zi & sign_mask)
                obuf[pl.ds(o, L)] = plsc.bitcast(gi, jnp.float32)
            return carry2

        lax.fori_loop(0, VPC // U, vec_body, 0)

    def chunk2(j2, carry):
        c0 = 2 * j2
        _in_copy(w_hbm, base, c0, in0, si0).wait()

        @pl.when(j2 > 0)
        def _():
            _out_copy(c0 - 2, out0, so0).wait()

        _xf(in0, out0)
        _out_copy(c0, out0, so0).start()

        @pl.when(c0 + 2 < NCHUNK)
        def _():
            _in_copy(w_hbm, base, c0 + 2, in0, si0).start()

        _in_copy(w_hbm, base, c0 + 1, in1, si1).wait()

        @pl.when(j2 > 0)
        def _():
            _out_copy(c0 - 1, out1, so1).wait()

        _xf(in1, out1)
        _out_copy(c0 + 1, out1, so1).start()

        @pl.when(c0 + 3 < NCHUNK)
        def _():
            _in_copy(w_hbm, base, c0 + 3, in1, si1).start()

        return carry

    lax.fori_loop(0, NCHUNK // 2, chunk2, 0)
    _out_copy(NCHUNK - 2, out0, so0).wait()
    _out_copy(NCHUNK - 1, out1, so1).wait()


@functools.partial(
    pl.kernel,
    out_type=jax.ShapeDtypeStruct((N,), jnp.float32),
    mesh=_MESH,
    compiler_params=pltpu.CompilerParams(needs_layout_passes=False),
    scratch_types=[
        pltpu.VMEM((2 * NW * L,), jnp.float32),
        pltpu.VMEM((L,), jnp.float32),
        pltpu.VMEM((L,), jnp.float32),
        pltpu.VMEM((L,), jnp.float32),
        pltpu.VMEM((CHUNK,), jnp.float32),
        pltpu.VMEM((CHUNK,), jnp.float32),
        pltpu.VMEM((CHUNK,), jnp.float32),
        pltpu.VMEM((CHUNK,), jnp.float32),
        pltpu.SemaphoreType.DMA,
        pltpu.SemaphoreType.DMA,
        pltpu.SemaphoreType.DMA,
        pltpu.SemaphoreType.DMA,
    ],
)
def _xform_call(w_hbm, psum_hbm, psq_hbm, table_hbm, alpha_hbm, out_hbm,
                pbuf, table_v, alpha_v, tmp_v,
                in0, in1, out0, out1, si0, si1, so0, so1):
    _xform_body(w_hbm, psum_hbm, psq_hbm, table_hbm, alpha_hbm, out_hbm,
                pbuf, table_v, alpha_v, tmp_v,
                in0, in1, out0, out1, si0, si1, so0, so1)


def kernel(weight, wgt_alpha):
    w = weight.reshape(N)
    grid = jnp.linspace(0.0, 1.0, 8, dtype=jnp.float32) * 1.0
    table = grid * wgt_alpha.astype(jnp.float32)
    table16 = jnp.concatenate([table, jnp.zeros((8,), jnp.float32)])
    alpha16 = jnp.full((L,), wgt_alpha, dtype=jnp.float32)
    psum, psq = _reduce_call(w)
    out = _xform_call(w, psum, psq, table16, alpha16)
    return out.reshape(R, C)


# speedup vs baseline: 3.7242x; 1.3003x over previous
import functools

import jax
import jax.numpy as jnp
from jax import lax
from jax.experimental import pallas as pl
from jax.experimental.pallas import tpu as pltpu
from jax.experimental.pallas import tpu_sc as plsc

NC = 2
NS = 16
L = 16
NW = NC * NS
R, C = 2048, 2048
N = R * C
PER_W = N // NW
CHUNK = 16384
NCHUNK = PER_W // CHUNK
VPC = CHUNK // L
U = 8

_ABS_MASK = 0x7FFFFFFF
_SIGN_MASK = -0x80000000

_MESH = plsc.VectorSubcoreMesh(
    core_axis_name="c", subcore_axis_name="s", num_cores=NC, num_subcores=NS
)


def _wid():
    return lax.axis_index("s") * NC + lax.axis_index("c")


def _lane_sum(v, tmp):
    lanes = lax.iota(jnp.int32, L)
    for s in (1, 2, 4, 8):
        tmp[...] = v
        v = v + plsc.load_gather(tmp, [lanes ^ s])
    return v


def _in_copy(w_hbm, base, c, buf, sem):
    return pltpu.make_async_copy(
        w_hbm.at[pl.ds(base + c * CHUNK, CHUNK)], buf, sem
    )


def _reduce_body(w_hbm, psum_hbm, psq_hbm, buf0, buf1, svec, qvec, sem0, sem1):
    base = _wid() * PER_W
    _in_copy(w_hbm, base, 0, buf0, sem0).start()
    _in_copy(w_hbm, base, 1, buf1, sem1).start()

    def _acc(buf, carry):
        def vec_body(o, carry2):
            s2, q2 = carry2
            xs = [buf[pl.ds(o + u * L, L)] for u in range(U)]
            for u in range(0, U, 2):
                s2 = s2 + (xs[u] + xs[u + 1])
                q2 = q2 + (xs[u] * xs[u] + xs[u + 1] * xs[u + 1])
            return s2, q2

        return plsc.parallel_loop(0, CHUNK, step=U * L, carry=carry)(vec_body)

    def chunk2(j2, carry):
        c0 = 2 * j2
        _in_copy(w_hbm, base, c0, buf0, sem0).wait()
        carry = _acc(buf0, carry)

        @pl.when(c0 + 2 < NCHUNK)
        def _():
            _in_copy(w_hbm, base, c0 + 2, buf0, sem0).start()

        _in_copy(w_hbm, base, c0 + 1, buf1, sem1).wait()
        carry = _acc(buf1, carry)

        @pl.when(c0 + 3 < NCHUNK)
        def _():
            _in_copy(w_hbm, base, c0 + 3, buf1, sem1).start()

        return carry

    zero = jnp.zeros((L,), jnp.float32)
    s, q = lax.fori_loop(0, NCHUNK // 2, chunk2, (zero, zero))
    svec[...] = s
    qvec[...] = q
    off = _wid() * L
    pltpu.sync_copy(svec, psum_hbm.at[pl.ds(off, L)])
    pltpu.sync_copy(qvec, psq_hbm.at[pl.ds(off, L)])


@functools.partial(
    pl.kernel,
    out_type=(
        jax.ShapeDtypeStruct((NW * L,), jnp.float32),
        jax.ShapeDtypeStruct((NW * L,), jnp.float32),
    ),
    mesh=_MESH,
    compiler_params=pltpu.CompilerParams(needs_layout_passes=False),
    scratch_types=[
        pltpu.VMEM((CHUNK,), jnp.float32),
        pltpu.VMEM((CHUNK,), jnp.float32),
        pltpu.VMEM((L,), jnp.float32),
        pltpu.VMEM((L,), jnp.float32),
        pltpu.SemaphoreType.DMA,
        pltpu.SemaphoreType.DMA,
    ],
)
def _reduce_call(w_hbm, psum_hbm, psq_hbm, buf0, buf1, svec, qvec, sem0, sem1):
    _reduce_body(w_hbm, psum_hbm, psq_hbm, buf0, buf1, svec, qvec, sem0, sem1)


def _xform_body(w_hbm, psum_hbm, psq_hbm, table_hbm, alpha_hbm, out_hbm,
                pbuf, table_v, alpha_v, tmp_v,
                in0, in1, out0, out1, si0, si1, so0, so1):
    base = _wid() * PER_W
    _in_copy(w_hbm, base, 0, in0, si0).start()
    _in_copy(w_hbm, base, 1, in1, si1).start()

    pltpu.sync_copy(psum_hbm, pbuf.at[pl.ds(0, NW * L)])
    pltpu.sync_copy(psq_hbm, pbuf.at[pl.ds(NW * L, NW * L)])
    pltpu.sync_copy(table_hbm, table_v)
    pltpu.sync_copy(alpha_hbm, alpha_v)

    def acc_body(i, carry):
        s, q = carry
        return (s + pbuf[pl.ds(i * L, L)],
                q + pbuf[pl.ds(NW * L + i * L, L)])

    z16 = jnp.zeros((L,), jnp.float32)
    s, q = lax.fori_loop(0, NW, acc_body, (z16, z16))
    tot = _lane_sum(s, tmp_v)
    totq = _lane_sum(q, tmp_v)
    mean = tot * jnp.float32(1.0 / N)
    var = (totq - jnp.float32(N) * mean * mean) * jnp.float32(1.0 / (N - 1))
    vb = plsc.bitcast(var, jnp.int32)
    magic = jnp.full((L,), 0x5F3759DF, dtype=jnp.int32)
    y = plsc.bitcast(magic - lax.shift_right_logical(vb, 1), jnp.float32)
    for _ in range(3):
        y = y * (jnp.float32(1.5) - jnp.float32(0.5) * var * y * y)
    s7 = (y / alpha_v[...]) * jnp.float32(7.0)
    m7 = mean * s7

    abs_mask = jnp.full((L,), _ABS_MASK, dtype=jnp.int32)
    sign_mask = jnp.full((L,), _SIGN_MASK, dtype=jnp.int32)
    seven_i = jnp.full((L,), 7, dtype=jnp.int32)
    seven_f = jnp.full((L,), 7.0, dtype=jnp.float32)
    half8 = jnp.full((L,), 7.5, dtype=jnp.float32)

    def _out_copy(c, buf, sem):
        return pltpu.make_async_copy(
            buf, out_hbm.at[pl.ds(base + c * CHUNK, CHUNK)], sem
        )

    def _xf(ibuf, obuf):
        def vec_body(o):
            x = ibuf[pl.ds(o, L)]
            z = x * s7 - m7
            zi = plsc.bitcast(z, jnp.int32)
            az = plsc.bitcast(zi & abs_mask, jnp.float32)
            az = jnp.minimum(az, seven_f)
            idx = seven_i - (half8 - az).astype(jnp.int32)
            g = plsc.load_gather(table_v, [idx])
            gi = plsc.bitcast(g, jnp.int32) ^ (zi & sign_mask)
            obuf[pl.ds(o, L)] = plsc.bitcast(gi, jnp.float32)

        plsc.parallel_loop(0, CHUNK, step=L, unroll=U)(vec_body)

    def chunk2(j2, carry):
        c0 = 2 * j2
        _in_copy(w_hbm, base, c0, in0, si0).wait()

        @pl.when(j2 > 0)
        def _():
            _out_copy(c0 - 2, out0, so0).wait()

        _xf(in0, out0)
        _out_copy(c0, out0, so0).start()

        @pl.when(c0 + 2 < NCHUNK)
        def _():
            _in_copy(w_hbm, base, c0 + 2, in0, si0).start()

        _in_copy(w_hbm, base, c0 + 1, in1, si1).wait()

        @pl.when(j2 > 0)
        def _():
            _out_copy(c0 - 1, out1, so1).wait()

        _xf(in1, out1)
        _out_copy(c0 + 1, out1, so1).start()

        @pl.when(c0 + 3 < NCHUNK)
        def _():
            _in_copy(w_hbm, base, c0 + 3, in1, si1).start()

        return carry

    lax.fori_loop(0, NCHUNK // 2, chunk2, 0)
    _out_copy(NCHUNK - 2, out0, so0).wait()
    _out_copy(NCHUNK - 1, out1, so1).wait()


@functools.partial(
    pl.kernel,
    out_type=jax.ShapeDtypeStruct((N,), jnp.float32),
    mesh=_MESH,
    compiler_params=pltpu.CompilerParams(needs_layout_passes=False),
    scratch_types=[
        pltpu.VMEM((2 * NW * L,), jnp.float32),
        pltpu.VMEM((L,), jnp.float32),
        pltpu.VMEM((L,), jnp.float32),
        pltpu.VMEM((L,), jnp.float32),
        pltpu.VMEM((CHUNK,), jnp.float32),
        pltpu.VMEM((CHUNK,), jnp.float32),
        pltpu.VMEM((CHUNK,), jnp.float32),
        pltpu.VMEM((CHUNK,), jnp.float32),
        pltpu.SemaphoreType.DMA,
        pltpu.SemaphoreType.DMA,
        pltpu.SemaphoreType.DMA,
        pltpu.SemaphoreType.DMA,
    ],
)
def _xform_call(w_hbm, psum_hbm, psq_hbm, table_hbm, alpha_hbm, out_hbm,
                pbuf, table_v, alpha_v, tmp_v,
                in0, in1, out0, out1, si0, si1, so0, so1):
    _xform_body(w_hbm, psum_hbm, psq_hbm, table_hbm, alpha_hbm, out_hbm,
                pbuf, table_v, alpha_v, tmp_v,
                in0, in1, out0, out1, si0, si1, so0, so1)


def kernel(weight, wgt_alpha):
    w = weight.reshape(N)
    grid = jnp.linspace(0.0, 1.0, 8, dtype=jnp.float32) * 1.0
    table = grid * wgt_alpha.astype(jnp.float32)
    table16 = jnp.concatenate([table, jnp.zeros((8,), jnp.float32)])
    alpha16 = jnp.full((L,), wgt_alpha, dtype=jnp.float32)
    psum, psq = _reduce_call(w)
    out = _xform_call(w, psum, psq, table16, alpha16)
    return out.reshape(R, C)


# speedup vs baseline: 5.2972x; 1.4224x over previous
import functools

import jax
import jax.numpy as jnp
from jax import lax
from jax.experimental import pallas as pl
from jax.experimental.pallas import tpu as pltpu
from jax.experimental.pallas import tpu_sc as plsc

NC = 2
NS = 16
L = 16
NW = NC * NS
R, C = 2048, 2048
N = R * C
ROWS_W = R // NW
CROWS = 8
NCHUNK = ROWS_W // CROWS
U = 8

_ABS_MASK = 0x7FFFFFFF
_SIGN_MASK = -0x80000000

_MESH = plsc.VectorSubcoreMesh(
    core_axis_name="c", subcore_axis_name="s", num_cores=NC, num_subcores=NS
)

_PARAMS = pltpu.CompilerParams(
    needs_layout_passes=False, use_tc_tiling_on_sc=True
)


def _wid():
    return lax.axis_index("s") * NC + lax.axis_index("c")


def _lane_sum(v, tmp):
    lanes = lax.iota(jnp.int32, L)
    for s in (1, 2, 4, 8):
        tmp[...] = v
        v = v + plsc.load_gather(tmp, [lanes ^ s])
    return v


def _in_copy(w_hbm, row0, c, buf, sem):
    return pltpu.make_async_copy(
        w_hbm.at[pl.ds(row0 + c * CROWS, CROWS), :], buf, sem
    )


def _acc(buf, carry):
    def row_sweep(o, carry2):
        s2, q2 = carry2
        for r in range(CROWS):
            xs = [buf[r, pl.ds(o + u * L, L)] for u in range(U)]
            for u in range(0, U, 2):
                s2 = s2 + (xs[u] + xs[u + 1])
                q2 = q2 + (xs[u] * xs[u] + xs[u + 1] * xs[u + 1])
        return s2, q2

    return plsc.parallel_loop(0, C, step=U * L, carry=carry)(row_sweep)


def _reduce_body(w_hbm, psum_hbm, psq_hbm, buf0, buf1, svec, qvec, sem0, sem1):
    row0 = _wid() * ROWS_W
    _in_copy(w_hbm, row0, 0, buf0, sem0).start()
    _in_copy(w_hbm, row0, 1, buf1, sem1).start()

    def chunk2(j2, carry):
        c0 = 2 * j2
        _in_copy(w_hbm, row0, c0, buf0, sem0).wait()
        carry = _acc(buf0, carry)

        @pl.when(c0 + 2 < NCHUNK)
        def _():
            _in_copy(w_hbm, row0, c0 + 2, buf0, sem0).start()

        _in_copy(w_hbm, row0, c0 + 1, buf1, sem1).wait()
        carry = _acc(buf1, carry)

        @pl.when(c0 + 3 < NCHUNK)
        def _():
            _in_copy(w_hbm, row0, c0 + 3, buf1, sem1).start()

        return carry

    zero = jnp.zeros((L,), jnp.float32)
    s, q = lax.fori_loop(0, NCHUNK // 2, chunk2, (zero, zero))
    svec[...] = s
    qvec[...] = q
    off = _wid() * L
    pltpu.sync_copy(svec, psum_hbm.at[pl.ds(off, L)])
    pltpu.sync_copy(qvec, psq_hbm.at[pl.ds(off, L)])


@functools.partial(
    pl.kernel,
    out_type=(
        jax.ShapeDtypeStruct((NW * L,), jnp.float32),
        jax.ShapeDtypeStruct((NW * L,), jnp.float32),
    ),
    mesh=_MESH,
    compiler_params=_PARAMS,
    scratch_types=[
        pltpu.VMEM((CROWS, C), jnp.float32),
        pltpu.VMEM((CROWS, C), jnp.float32),
        pltpu.VMEM((L,), jnp.float32),
        pltpu.VMEM((L,), jnp.float32),
        pltpu.SemaphoreType.DMA,
        pltpu.SemaphoreType.DMA,
    ],
)
def _reduce_call(w_hbm, psum_hbm, psq_hbm, buf0, buf1, svec, qvec, sem0, sem1):
    _reduce_body(w_hbm, psum_hbm, psq_hbm, buf0, buf1, svec, qvec, sem0, sem1)


def _xform_body(w_hbm, psum_hbm, psq_hbm, table_hbm, alpha_hbm, out_hbm,
                pbuf, table_v, alpha_v, tmp_v,
                in0, in1, out0, out1, si0, si1, so0, so1):
    row0 = _wid() * ROWS_W
    _in_copy(w_hbm, row0, 0, in0, si0).start()
    _in_copy(w_hbm, row0, 1, in1, si1).start()

    pltpu.sync_copy(psum_hbm, pbuf.at[pl.ds(0, NW * L)])
    pltpu.sync_copy(psq_hbm, pbuf.at[pl.ds(NW * L, NW * L)])
    pltpu.sync_copy(table_hbm, table_v)
    pltpu.sync_copy(alpha_hbm, alpha_v)

    def acc_body(i, carry):
        s, q = carry
        return (s + pbuf[pl.ds(i * L, L)],
                q + pbuf[pl.ds(NW * L + i * L, L)])

    z16 = jnp.zeros((L,), jnp.float32)
    s, q = lax.fori_loop(0, NW, acc_body, (z16, z16))
    tot = _lane_sum(s, tmp_v)
    totq = _lane_sum(q, tmp_v)
    mean = tot * jnp.float32(1.0 / N)
    var = (totq - jnp.float32(N) * mean * mean) * jnp.float32(1.0 / (N - 1))
    vb = plsc.bitcast(var, jnp.int32)
    magic = jnp.full((L,), 0x5F3759DF, dtype=jnp.int32)
    y = plsc.bitcast(magic - lax.shift_right_logical(vb, 1), jnp.float32)
    for _ in range(3):
        y = y * (jnp.float32(1.5) - jnp.float32(0.5) * var * y * y)
    s7 = (y / alpha_v[...]) * jnp.float32(7.0)
    m7 = mean * s7

    abs_mask = jnp.full((L,), _ABS_MASK, dtype=jnp.int32)
    sign_mask = jnp.full((L,), _SIGN_MASK, dtype=jnp.int32)
    seven_i = jnp.full((L,), 7, dtype=jnp.int32)
    seven_f = jnp.full((L,), 7.0, dtype=jnp.float32)
    half8 = jnp.full((L,), 7.5, dtype=jnp.float32)

    def _out_copy(c, buf, sem):
        return pltpu.make_async_copy(
            buf, out_hbm.at[pl.ds(row0 + c * CROWS, CROWS), :], sem
        )

    def _xf(ibuf, obuf):
        def vec_body(o):
            for r in range(CROWS):
                x = ibuf[r, pl.ds(o, L)]
                z = x * s7 - m7
                zi = plsc.bitcast(z, jnp.int32)
                az = plsc.bitcast(zi & abs_mask, jnp.float32)
                az = jnp.minimum(az, seven_f)
                idx = seven_i - (half8 - az).astype(jnp.int32)
                g = plsc.load_gather(table_v, [idx])
                gi = plsc.bitcast(g, jnp.int32) ^ (zi & sign_mask)
                obuf[r, pl.ds(o, L)] = plsc.bitcast(gi, jnp.float32)

        plsc.parallel_loop(0, C, step=L, unroll=U)(vec_body)

    def chunk2(j2, carry):
        c0 = 2 * j2
        _in_copy(w_hbm, row0, c0, in0, si0).wait()

        @pl.when(j2 > 0)
        def _():
            _out_copy(c0 - 2, out0, so0).wait()

        _xf(in0, out0)
        _out_copy(c0, out0, so0).start()

        @pl.when(c0 + 2 < NCHUNK)
        def _():
            _in_copy(w_hbm, row0, c0 + 2, in0, si0).start()

        _in_copy(w_hbm, row0, c0 + 1, in1, si1).wait()

        @pl.when(j2 > 0)
        def _():
            _out_copy(c0 - 1, out1, so1).wait()

        _xf(in1, out1)
        _out_copy(c0 + 1, out1, so1).start()

        @pl.when(c0 + 3 < NCHUNK)
        def _():
            _in_copy(w_hbm, row0, c0 + 3, in1, si1).start()

        return carry

    lax.fori_loop(0, NCHUNK // 2, chunk2, 0)
    _out_copy(NCHUNK - 2, out0, so0).wait()
    _out_copy(NCHUNK - 1, out1, so1).wait()


@functools.partial(
    pl.kernel,
    out_type=jax.ShapeDtypeStruct((R, C), jnp.float32),
    mesh=_MESH,
    compiler_params=_PARAMS,
    scratch_types=[
        pltpu.VMEM((2 * NW * L,), jnp.float32),
        pltpu.VMEM((L,), jnp.float32),
        pltpu.VMEM((L,), jnp.float32),
        pltpu.VMEM((L,), jnp.float32),
        pltpu.VMEM((CROWS, C), jnp.float32),
        pltpu.VMEM((CROWS, C), jnp.float32),
        pltpu.VMEM((CROWS, C), jnp.float32),
        pltpu.VMEM((CROWS, C), jnp.float32),
        pltpu.SemaphoreType.DMA,
        pltpu.SemaphoreType.DMA,
        pltpu.SemaphoreType.DMA,
        pltpu.SemaphoreType.DMA,
    ],
)
def _xform_call(w_hbm, psum_hbm, psq_hbm, table_hbm, alpha_hbm, out_hbm,
                pbuf, table_v, alpha_v, tmp_v,
                in0, in1, out0, out1, si0, si1, so0, so1):
    _xform_body(w_hbm, psum_hbm, psq_hbm, table_hbm, alpha_hbm, out_hbm,
                pbuf, table_v, alpha_v, tmp_v,
                in0, in1, out0, out1, si0, si1, so0, so1)


def kernel(weight, wgt_alpha):
    grid = jnp.linspace(0.0, 1.0, 8, dtype=jnp.float32) * 1.0
    table = grid * wgt_alpha.astype(jnp.float32)
    table16 = jnp.concatenate([table, jnp.zeros((8,), jnp.float32)])
    alpha16 = jnp.full((L,), wgt_alpha, dtype=jnp.float32)
    psum, psq = _reduce_call(weight)
    out = _xform_call(weight, psum, psq, table16, alpha16)
    return out


# speedup vs baseline: 5.4277x; 1.0246x over previous
import functools

import jax
import jax.numpy as jnp
from jax import lax
from jax.experimental import pallas as pl
from jax.experimental.pallas import tpu as pltpu
from jax.experimental.pallas import tpu_sc as plsc

NC = 2
NS = 16
L = 16
NW = NC * NS
R, C = 2048, 2048
N = R * C
ROWS_W = R // NW
CROWS = 8
NCHUNK = ROWS_W // CROWS
XROWS = 4
XNCH = ROWS_W // XROWS
XD = 4
U = 8

_ABS_MASK = 0x7FFFFFFF
_SIGN_MASK = -0x80000000

_MESH = plsc.VectorSubcoreMesh(
    core_axis_name="c", subcore_axis_name="s", num_cores=NC, num_subcores=NS
)

_PARAMS = pltpu.CompilerParams(
    needs_layout_passes=False, use_tc_tiling_on_sc=True
)


def _wid():
    return lax.axis_index("s") * NC + lax.axis_index("c")


def _lane_sum(v, tmp):
    lanes = lax.iota(jnp.int32, L)
    for s in (1, 2, 4, 8):
        tmp[...] = v
        v = v + plsc.load_gather(tmp, [lanes ^ s])
    return v


def _in_copy(w_hbm, row0, c, buf, sem):
    return pltpu.make_async_copy(
        w_hbm.at[pl.ds(row0 + c * CROWS, CROWS), :], buf, sem
    )


def _acc(buf, carry):
    def row_sweep(o, carry2):
        s2, q2 = carry2
        for r in range(CROWS):
            xs = [buf[r, pl.ds(o + u * L, L)] for u in range(U)]
            for u in range(0, U, 2):
                s2 = s2 + (xs[u] + xs[u + 1])
                q2 = q2 + (xs[u] * xs[u] + xs[u + 1] * xs[u + 1])
        return s2, q2

    return plsc.parallel_loop(0, C, step=U * L, carry=carry)(row_sweep)


def _reduce_body(w_hbm, psum_hbm, psq_hbm, buf0, buf1, svec, qvec, sem0, sem1):
    row0 = _wid() * ROWS_W
    _in_copy(w_hbm, row0, 0, buf0, sem0).start()
    _in_copy(w_hbm, row0, 1, buf1, sem1).start()

    def chunk2(j2, carry):
        c0 = 2 * j2
        _in_copy(w_hbm, row0, c0, buf0, sem0).wait()
        carry = _acc(buf0, carry)

        @pl.when(c0 + 2 < NCHUNK)
        def _():
            _in_copy(w_hbm, row0, c0 + 2, buf0, sem0).start()

        _in_copy(w_hbm, row0, c0 + 1, buf1, sem1).wait()
        carry = _acc(buf1, carry)

        @pl.when(c0 + 3 < NCHUNK)
        def _():
            _in_copy(w_hbm, row0, c0 + 3, buf1, sem1).start()

        return carry

    zero = jnp.zeros((L,), jnp.float32)
    s, q = lax.fori_loop(0, NCHUNK // 2, chunk2, (zero, zero))
    svec[...] = s
    qvec[...] = q
    off = _wid() * L
    pltpu.sync_copy(svec, psum_hbm.at[pl.ds(off, L)])
    pltpu.sync_copy(qvec, psq_hbm.at[pl.ds(off, L)])


@functools.partial(
    pl.kernel,
    out_type=(
        jax.ShapeDtypeStruct((NW * L,), jnp.float32),
        jax.ShapeDtypeStruct((NW * L,), jnp.float32),
    ),
    mesh=_MESH,
    compiler_params=_PARAMS,
    scratch_types=[
        pltpu.VMEM((CROWS, C), jnp.float32),
        pltpu.VMEM((CROWS, C), jnp.float32),
        pltpu.VMEM((L,), jnp.float32),
        pltpu.VMEM((L,), jnp.float32),
        pltpu.SemaphoreType.DMA,
        pltpu.SemaphoreType.DMA,
    ],
)
def _reduce_call(w_hbm, psum_hbm, psq_hbm, buf0, buf1, svec, qvec, sem0, sem1):
    _reduce_body(w_hbm, psum_hbm, psq_hbm, buf0, buf1, svec, qvec, sem0, sem1)


def _xform_body(w_hbm, psum_hbm, psq_hbm, table_hbm, alpha_hbm, out_hbm,
                pbuf, table_v, alpha_v, tmp_v,
                in0, in1, in2, in3, out0, out1, out2, out3,
                si0, si1, si2, si3, so0, so1, so2, so3):
    row0 = _wid() * ROWS_W

    def _xin_copy(c, buf, sem):
        return pltpu.make_async_copy(
            w_hbm.at[pl.ds(row0 + c * XROWS, XROWS), :], buf, sem
        )

    ins = [in0, in1, in2, in3]
    outs = [out0, out1, out2, out3]
    isems = [si0, si1, si2, si3]
    osems = [so0, so1, so2, so3]
    for b in range(XD):
        _xin_copy(b, ins[b], isems[b]).start()

    pltpu.sync_copy(psum_hbm, pbuf.at[pl.ds(0, NW * L)])
    pltpu.sync_copy(psq_hbm, pbuf.at[pl.ds(NW * L, NW * L)])
    pltpu.sync_copy(table_hbm, table_v)
    pltpu.sync_copy(alpha_hbm, alpha_v)

    def acc_body(i, carry):
        s, q = carry
        return (s + pbuf[pl.ds(i * L, L)],
                q + pbuf[pl.ds(NW * L + i * L, L)])

    z16 = jnp.zeros((L,), jnp.float32)
    s, q = lax.fori_loop(0, NW, acc_body, (z16, z16))
    tot = _lane_sum(s, tmp_v)
    totq = _lane_sum(q, tmp_v)
    mean = tot * jnp.float32(1.0 / N)
    var = (totq - jnp.float32(N) * mean * mean) * jnp.float32(1.0 / (N - 1))
    vb = plsc.bitcast(var, jnp.int32)
    magic = jnp.full((L,), 0x5F3759DF, dtype=jnp.int32)
    y = plsc.bitcast(magic - lax.shift_right_logical(vb, 1), jnp.float32)
    for _ in range(3):
        y = y * (jnp.float32(1.5) - jnp.float32(0.5) * var * y * y)
    s7 = (y / alpha_v[...]) * jnp.float32(7.0)
    m7 = mean * s7

    abs_mask = jnp.full((L,), _ABS_MASK, dtype=jnp.int32)
    sign_mask = jnp.full((L,), _SIGN_MASK, dtype=jnp.int32)
    seven_i = jnp.full((L,), 7, dtype=jnp.int32)
    seven_f = jnp.full((L,), 7.0, dtype=jnp.float32)
    half8 = jnp.full((L,), 7.5, dtype=jnp.float32)

    def _out_copy(c, buf, sem):
        return pltpu.make_async_copy(
            buf, out_hbm.at[pl.ds(row0 + c * XROWS, XROWS), :], sem
        )

    def _xf(ibuf, obuf):
        def vec_body(o):
            for r in range(XROWS):
                x = ibuf[r, pl.ds(o, L)]
                z = x * s7 - m7
                zi = plsc.bitcast(z, jnp.int32)
                az = plsc.bitcast(zi & abs_mask, jnp.float32)
                az = jnp.minimum(az, seven_f)
                idx = seven_i - (half8 - az).astype(jnp.int32)
                g = plsc.load_gather(table_v, [idx])
                gi = plsc.bitcast(g, jnp.int32) ^ (zi & sign_mask)
                obuf[r, pl.ds(o, L)] = plsc.bitcast(gi, jnp.float32)

        plsc.parallel_loop(0, C, step=L, unroll=U)(vec_body)

    def ring(j4, carry):
        for b in range(XD):
            c = XD * j4 + b
            _xin_copy(c, ins[b], isems[b]).wait()

            @pl.when(j4 > 0)
            def _():
                _out_copy(c - XD, outs[b], osems[b]).wait()

            _xf(ins[b], outs[b])
            _out_copy(c, outs[b], osems[b]).start()

            @pl.when(c + XD < XNCH)
            def _():
                _xin_copy(c + XD, ins[b], isems[b]).start()

        return carry

    lax.fori_loop(0, XNCH // XD, ring, 0)
    for b in range(XD):
        _out_copy(XNCH - XD + b, outs[b], osems[b]).wait()


@functools.partial(
    pl.kernel,
    out_type=jax.ShapeDtypeStruct((R, C), jnp.float32),
    mesh=_MESH,
    compiler_params=_PARAMS,
    scratch_types=[
        pltpu.VMEM((2 * NW * L,), jnp.float32),
        pltpu.VMEM((L,), jnp.float32),
        pltpu.VMEM((L,), jnp.float32),
        pltpu.VMEM((L,), jnp.float32),
    ]
    + [pltpu.VMEM((XROWS, C), jnp.float32)] * (2 * XD)
    + [pltpu.SemaphoreType.DMA] * (2 * XD),
)
def _xform_call(w_hbm, psum_hbm, psq_hbm, table_hbm, alpha_hbm, out_hbm,
                pbuf, table_v, alpha_v, tmp_v,
                in0, in1, in2, in3, out0, out1, out2, out3,
                si0, si1, si2, si3, so0, so1, so2, so3):
    _xform_body(w_hbm, psum_hbm, psq_hbm, table_hbm, alpha_hbm, out_hbm,
                pbuf, table_v, alpha_v, tmp_v,
                in0, in1, in2, in3, out0, out1, out2, out3,
                si0, si1, si2, si3, so0, so1, so2, so3)


def kernel(weight, wgt_alpha):
    grid = jnp.linspace(0.0, 1.0, 8, dtype=jnp.float32) * 1.0
    table = grid * wgt_alpha.astype(jnp.float32)
    table16 = jnp.concatenate([table, jnp.zeros((8,), jnp.float32)])
    alpha16 = jnp.full((L,), wgt_alpha, dtype=jnp.float32)
    psum, psq = _reduce_call(weight)
    out = _xform_call(weight, psum, psq, table16, alpha16)
    return out


# speedup vs baseline: 5.6135x; 1.0342x over previous
import functools

import jax
import jax.numpy as jnp
from jax import lax
from jax.experimental import pallas as pl
from jax.experimental.pallas import tpu as pltpu
from jax.experimental.pallas import tpu_sc as plsc

NC = 2
NS = 16
L = 16
NW = NC * NS
R, C = 2048, 2048
N = R * C
ROWS_W = R // NW
CROWS = 8
NCHUNK = ROWS_W // CROWS
XROWS = 4
XNCH = ROWS_W // XROWS
XD = 4
U = 8

_ABS_MASK = 0x7FFFFFFF
_SIGN_MASK = -0x80000000

_MESH = plsc.VectorSubcoreMesh(
    core_axis_name="c", subcore_axis_name="s", num_cores=NC, num_subcores=NS
)

_PARAMS = pltpu.CompilerParams(
    needs_layout_passes=False, use_tc_tiling_on_sc=True
)


def _wid():
    return lax.axis_index("s") * NC + lax.axis_index("c")


TCROWS = 128


def _tc_stats_body(w_ref, stat_ref, acc_ref):
    i = pl.program_id(0)
    x = w_ref[...]
    s = jnp.sum(x)
    q = jnp.sum(x * x)

    @pl.when(i == 0)
    def _():
        acc_ref[0] = jnp.float32(0.0)
        acc_ref[1] = jnp.float32(0.0)

    acc_ref[0] = acc_ref[0] + s
    acc_ref[1] = acc_ref[1] + q

    @pl.when(i == pl.num_programs(0) - 1)
    def _():
        for j in range(2, L):
            stat_ref[j] = jnp.float32(0.0)
        stat_ref[0] = acc_ref[0]
        stat_ref[1] = acc_ref[1]


_tc_stats = pl.pallas_call(
    _tc_stats_body,
    grid=(R // TCROWS,),
    in_specs=[pl.BlockSpec((TCROWS, C), lambda i: (i, 0))],
    out_specs=pl.BlockSpec(memory_space=pltpu.SMEM),
    out_shape=jax.ShapeDtypeStruct((L,), jnp.float32),
    scratch_shapes=[pltpu.SMEM((2,), jnp.float32)],
    compiler_params=pltpu.CompilerParams(dimension_semantics=("arbitrary",)),
)


def _xform_body(w_hbm, stats_hbm, table_hbm, alpha_hbm, out_hbm,
                stat_v, table_v, alpha_v,
                in0, in1, in2, in3, out0, out1, out2, out3,
                si0, si1, si2, si3, so0, so1, so2, so3):
    row0 = _wid() * ROWS_W

    def _xin_copy(c, buf, sem):
        return pltpu.make_async_copy(
            w_hbm.at[pl.ds(row0 + c * XROWS, XROWS), :], buf, sem
        )

    ins = [in0, in1, in2, in3]
    outs = [out0, out1, out2, out3]
    isems = [si0, si1, si2, si3]
    osems = [so0, so1, so2, so3]
    for b in range(XD):
        _xin_copy(b, ins[b], isems[b]).start()

    pltpu.sync_copy(stats_hbm, stat_v)
    pltpu.sync_copy(table_hbm, table_v)
    pltpu.sync_copy(alpha_hbm, alpha_v)

    zero_i = jnp.zeros((L,), jnp.int32)
    tot = plsc.load_gather(stat_v, [zero_i])
    totq = plsc.load_gather(stat_v, [zero_i + 1])
    mean = tot * jnp.float32(1.0 / N)
    var = (totq - jnp.float32(N) * mean * mean) * jnp.float32(1.0 / (N - 1))
    vb = plsc.bitcast(var, jnp.int32)
    magic = jnp.full((L,), 0x5F3759DF, dtype=jnp.int32)
    y = plsc.bitcast(magic - lax.shift_right_logical(vb, 1), jnp.float32)
    for _ in range(3):
        y = y * (jnp.float32(1.5) - jnp.float32(0.5) * var * y * y)
    s7 = (y / alpha_v[...]) * jnp.float32(7.0)
    m7 = mean * s7

    abs_mask = jnp.full((L,), _ABS_MASK, dtype=jnp.int32)
    sign_mask = jnp.full((L,), _SIGN_MASK, dtype=jnp.int32)
    seven_i = jnp.full((L,), 7, dtype=jnp.int32)
    seven_f = jnp.full((L,), 7.0, dtype=jnp.float32)
    half8 = jnp.full((L,), 7.5, dtype=jnp.float32)

    def _out_copy(c, buf, sem):
        return pltpu.make_async_copy(
            buf, out_hbm.at[pl.ds(row0 + c * XROWS, XROWS), :], sem
        )

    def _xf(ibuf, obuf):
        def vec_body(o):
            for r in range(XROWS):
                x = ibuf[r, pl.ds(o, L)]
                z = x * s7 - m7
                zi = plsc.bitcast(z, jnp.int32)
                az = plsc.bitcast(zi & abs_mask, jnp.float32)
                az = jnp.minimum(az, seven_f)
                idx = seven_i - (half8 - az).astype(jnp.int32)
                g = plsc.load_gather(table_v, [idx])
                gi = plsc.bitcast(g, jnp.int32) ^ (zi & sign_mask)
                obuf[r, pl.ds(o, L)] = plsc.bitcast(gi, jnp.float32)

        plsc.parallel_loop(0, C, step=L, unroll=U)(vec_body)

    def ring(j4, carry):
        for b in range(XD):
            c = XD * j4 + b
            _xin_copy(c, ins[b], isems[b]).wait()

            @pl.when(j4 > 0)
            def _():
                _out_copy(c - XD, outs[b], osems[b]).wait()

            _xf(ins[b], outs[b])
            _out_copy(c, outs[b], osems[b]).start()

            @pl.when(c + XD < XNCH)
            def _():
                _xin_copy(c + XD, ins[b], isems[b]).start()

        return carry

    lax.fori_loop(0, XNCH // XD, ring, 0)
    for b in range(XD):
        _out_copy(XNCH - XD + b, outs[b], osems[b]).wait()


@functools.partial(
    pl.kernel,
    out_type=jax.ShapeDtypeStruct((R, C), jnp.float32),
    mesh=_MESH,
    compiler_params=_PARAMS,
    scratch_types=[
        pltpu.VMEM((L,), jnp.float32),
        pltpu.VMEM((L,), jnp.float32),
        pltpu.VMEM((L,), jnp.float32),
    ]
    + [pltpu.VMEM((XROWS, C), jnp.float32)] * (2 * XD)
    + [pltpu.SemaphoreType.DMA] * (2 * XD),
)
def _xform_call(w_hbm, stats_hbm, table_hbm, alpha_hbm, out_hbm,
                stat_v, table_v, alpha_v,
                in0, in1, in2, in3, out0, out1, out2, out3,
                si0, si1, si2, si3, so0, so1, so2, so3):
    _xform_body(w_hbm, stats_hbm, table_hbm, alpha_hbm, out_hbm,
                stat_v, table_v, alpha_v,
                in0, in1, in2, in3, out0, out1, out2, out3,
                si0, si1, si2, si3, so0, so1, so2, so3)


def kernel(weight, wgt_alpha):
    grid = jnp.linspace(0.0, 1.0, 8, dtype=jnp.float32) * 1.0
    table = grid * wgt_alpha.astype(jnp.float32)
    table16 = jnp.concatenate([table, jnp.zeros((8,), jnp.float32)])
    alpha16 = jnp.full((L,), wgt_alpha, dtype=jnp.float32)
    stats = _tc_stats(weight)
    out = _xform_call(weight, stats, table16, alpha16)
    return out
